# baseline (speedup 1.0000x reference)
import functools

import jax
import jax.numpy as jnp
from jax import lax
from jax.experimental import pallas as pl
from jax.experimental.pallas import tpu as pltpu
from jax.experimental.pallas import tpu_sc as plsc

_N_CORES = 2
_N_SUBCORES = 16
_NW = _N_CORES * _N_SUBCORES
_LANES = 16

_CHUNK = 40
_NBUF = 2
_GBLK = 25
_DROWS = 200


def _make_sc_partials(N, D, E):
    ew = E // _NW
    nchunk = ew // _CHUNK
    nblk = nchunk // _GBLK
    nzc = N // _CHUNK
    ndc = N // _DROWS
    nsl = D // _LANES

    def body(src_hbm, dst_hbm, node_hbm, ef_hbm, part_hbm,
             sidx_v, didx_v, rows_v, ef_v, msg_v, acc_sh,
             sem_g, sem_e, sem_s, sem_i, sem_z):
        cid = lax.axis_index("c")
        sid = lax.axis_index("s")
        wid = sid * _N_CORES + cid

        def ef_slice(c):
            base = pl.multiple_of(wid * ew + c * _CHUNK, 8)
            return ef_hbm.at[pl.ds(base, _CHUNK)]

        def idx_refs(c):
            blk = c // _GBLK
            bb = lax.rem(blk, 2)
            cc = lax.rem(c, _GBLK)
            return sidx_v.at[bb, cc], didx_v.at[bb, cc]

        def issue(c, b):
            si, _ = idx_refs(c)
            pltpu.async_copy(node_hbm.at[si], rows_v.at[b], sem_g[b])
            pltpu.async_copy(ef_slice(c), ef_v.at[b], sem_e[b])

        def wait_loads(c, b):
            si, _ = idx_refs(c)
            pltpu.make_async_copy(
                node_hbm.at[si], rows_v.at[b], sem_g[b]).wait()
            pltpu.make_async_copy(
                ef_slice(c), ef_v.at[b], sem_e[b]).wait()

        def compute(b):
            def row(i, _):
                for j in range(nsl):
                    sl = pl.ds(j * _LANES, _LANES)
                    msg_v[b, i, sl] = jnp.maximum(
                        rows_v[b, i, sl] + ef_v[b, i, sl], 0.0)
                return 0
            lax.fori_loop(0, _CHUNK, row, 0)

        def scatter(c, b):
            _, di = idx_refs(c)
            pltpu.async_copy(msg_v.at[b], acc_sh.at[di], sem_s[b], add=True)

        def wait_scatter(b):
            pltpu.make_async_copy(
                msg_v.at[b], acc_sh.at[didx_v.at[0, 0]], sem_s[b]).wait()

        def issue_idx_block(blk):
            bb = lax.rem(blk, 2)
            pltpu.async_copy(src_hbm.at[wid, blk], sidx_v.at[bb], sem_i)
            pltpu.async_copy(dst_hbm.at[wid, blk], didx_v.at[bb], sem_i)

        def wait_idx_block(blk):
            bb = lax.rem(blk, 2)
            pltpu.make_async_copy(
                src_hbm.at[wid, blk], sidx_v.at[bb], sem_i).wait()
            pltpu.make_async_copy(
                dst_hbm.at[wid, blk], didx_v.at[bb], sem_i).wait()

        issue_idx_block(0)
        wait_idx_block(0)
        for b in range(_NBUF):
            issue(b, b)

        def zrow(i, _):
            for j in range(nsl):
                msg_v[0, i, pl.ds(j * _LANES, _LANES)] = jnp.zeros(
                    (_LANES,), jnp.float32)
            return 0
        lax.fori_loop(0, _CHUNK, zrow, 0)
        nzt = (nzc + _N_SUBCORES - 1) // _N_SUBCORES
        for t in range(nzt):
            j = t * _N_SUBCORES + sid

            @pl.when(j < nzc)
            def _():
                r0 = pl.multiple_of(j * _CHUNK, 8)
                pltpu.async_copy(msg_v.at[0], acc_sh.at[pl.ds(r0, _CHUNK)],
                                 sem_z)
        for t in range(nzt):
            j = t * _N_SUBCORES + sid

            @pl.when(j < nzc)
            def _():
                pltpu.make_async_copy(
                    msg_v.at[0], acc_sh.at[pl.ds(0, _CHUNK)], sem_z).wait()
        plsc.subcore_barrier()

        def pair(g, _):
            for b in range(_NBUF):
                c = g * _NBUF + b
                cc = lax.rem(c, _GBLK)
                blk = c // _GBLK
                wait_loads(c, b)

                @pl.when(c >= _NBUF)
                def _():
                    wait_scatter(b)
                compute(b)

                @pl.when(jnp.logical_and(cc == 4, blk + 1 < nblk))
                def _():
                    issue_idx_block(blk + 1)

                @pl.when(jnp.logical_and(cc == _GBLK - 2, blk + 1 < nblk))
                def _():
                    wait_idx_block(blk + 1)

                @pl.when(c + _NBUF < nchunk)
                def _():
                    issue(c + _NBUF, b)
                scatter(c, b)
            return 0
        lax.fori_loop(0, nchunk // _NBUF, pair, 0)
        for b in range(_NBUF):
            wait_scatter(b)

        plsc.subcore_barrier()
        for t in range((ndc + _N_SUBCORES - 1) // _N_SUBCORES):
            j = t * _N_SUBCORES + sid

            @pl.when(j < ndc)
            def _():
                r0 = pl.multiple_of(j * _DROWS, 8)
                pltpu.sync_copy(acc_sh.at[pl.ds(r0, _DROWS)],
                                part_hbm.at[cid, pl.ds(r0, _DROWS)])

    mesh = plsc.VectorSubcoreMesh(core_axis_name="c", subcore_axis_name="s")
    return pl.kernel(
        body,
        out_type=jax.ShapeDtypeStruct((_N_CORES, N, D), jnp.float32),
        mesh=mesh,
        scratch_types=[
            pltpu.VMEM((2, _GBLK, _CHUNK), jnp.int32),
            pltpu.VMEM((2, _GBLK, _CHUNK), jnp.int32),
            pltpu.VMEM((_NBUF, _CHUNK, D), jnp.float32),
            pltpu.VMEM((_NBUF, _CHUNK, D), jnp.float32),
            pltpu.VMEM((_NBUF, _CHUNK, D), jnp.float32),
            pltpu.VMEM_SHARED((N, D), jnp.float32),
            [pltpu.SemaphoreType.DMA] * _NBUF,
            [pltpu.SemaphoreType.DMA] * _NBUF,
            [pltpu.SemaphoreType.DMA] * _NBUF,
            pltpu.SemaphoreType.DMA,
            pltpu.SemaphoreType.DMA,
        ],
    )


def _combine_body(eps_ref, x_ref, p0_ref, p1_ref, o_ref):
    o_ref[...] = (x_ref[...] * (1.0 + eps_ref[0])
                  + p0_ref[...] + p1_ref[...])


def _combine(eps, node_feat, p0, p1):
    N, D = node_feat.shape
    br = 1000
    return pl.pallas_call(
        _combine_body,
        out_shape=jax.ShapeDtypeStruct((N, D), jnp.float32),
        grid=(N // br,),
        in_specs=[
            pl.BlockSpec(memory_space=pltpu.SMEM),
            pl.BlockSpec((br, D), lambda i: (i, 0)),
            pl.BlockSpec((br, D), lambda i: (i, 0)),
            pl.BlockSpec((br, D), lambda i: (i, 0)),
        ],
        out_specs=pl.BlockSpec((br, D), lambda i: (i, 0)),
    )(eps, node_feat, p0, p1)


def kernel(node_feat, edge_index, edge_feat, eps):
    N, D = node_feat.shape
    E = edge_feat.shape[0]
    ew = E // _NW
    nchunk = ew // _CHUNK
    nblk = nchunk // _GBLK
    src = edge_index[0].astype(jnp.int32).reshape(_NW, nblk, _GBLK, _CHUNK)
    dst = edge_index[1].astype(jnp.int32).reshape(_NW, nblk, _GBLK, _CHUNK)
    partials = _make_sc_partials(N, D, E)(src, dst, node_feat, edge_feat)
    return _combine(eps.astype(jnp.float32), node_feat,
                    partials[0], partials[1])
